# Initial kernel scaffold; baseline (speedup 1.0000x reference)
import functools

import jax
import jax.numpy as jnp
from jax import lax
from jax.experimental import pallas as pl
from jax.experimental.pallas import tpu as pltpu
from jax.experimental.pallas import tpu_sc as plsc

N_V = 10000
N_E = 5000
NNZ = 320000
D = 128
DA = 144
E_PAD = 5120
V_PAD = 10240
NC = 2
NS = 16
NW = NC * NS
C = 80
KCH = NNZ // (NW * C)
L = 16


def _tc_matmul(X, W, b):
    blk = 400

    def body(x_ref, w_ref, b_ref, o_ref):
        y = lax.dot_general(x_ref[...], w_ref[...], (((1,), (1,)), ((), ())),
                            preferred_element_type=jnp.float32)
        y = jnp.maximum(y + b_ref[...], 0.0)
        aug = (lax.broadcasted_iota(jnp.int32, (blk, DA - D), 1) == 0)
        o_ref[...] = jnp.concatenate([y, aug.astype(jnp.float32)], axis=1)

    return pl.pallas_call(
        body,
        grid=(N_V // blk,),
        in_specs=[
            pl.BlockSpec((blk, D), lambda i: (i, 0)),
            pl.BlockSpec((D, D), lambda i: (0, 0)),
            pl.BlockSpec((1, D), lambda i: (0, 0)),
        ],
        out_specs=pl.BlockSpec((blk, DA), lambda i: (i, 0)),
        out_shape=jax.ShapeDtypeStruct((N_V, DA), jnp.float32),
    )(X, W, b.reshape(1, D))


def _make_seg_sum(n_pad):
    rows_per_tile = n_pad // NS
    n_blk = rows_per_tile // C
    mesh = plsc.VectorSubcoreMesh(core_axis_name="c", subcore_axis_name="s")

    @functools.partial(
        pl.kernel,
        out_type=jax.ShapeDtypeStruct((NC, n_pad, DA), jnp.float32),
        mesh=mesh,
        scratch_types=[
            pltpu.VMEM((KCH, C), jnp.int32),
            pltpu.VMEM((KCH, C), jnp.int32),
            pltpu.VMEM((C, DA), jnp.float32),
            pltpu.VMEM_SHARED((n_pad, DA), jnp.float32),
            pltpu.SemaphoreType.DMA,
        ],
    )
    def seg(src_hbm, gidx_hbm, sidx_hbm, out_hbm, gidx_v, sidx_v, rows_v,
            acc_sh, sem):
        c = lax.axis_index("c")
        s = lax.axis_index("s")
        wid = c * NS + s

        def zero_row(r, carry):
            for g in range(DA // L):
                rows_v[r, pl.ds(g * L, L)] = jnp.zeros((L,), jnp.float32)
            return carry
        lax.fori_loop(0, C, zero_row, 0)
        base = s * rows_per_tile
        for j in range(n_blk):
            pltpu.sync_copy(rows_v, acc_sh.at[pl.ds(base + j * C, C)])
        plsc.subcore_barrier()

        pltpu.sync_copy(gidx_hbm.at[pl.ds(wid * KCH, KCH)], gidx_v)
        pltpu.sync_copy(sidx_hbm.at[pl.ds(wid * KCH, KCH)], sidx_v)

        def chunk(k, carry):
            pltpu.async_copy(src_hbm.at[gidx_v.at[k]], rows_v, sem).wait()
            pltpu.sync_copy(rows_v, acc_sh.at[sidx_v.at[k]], add=True)
            return carry
        lax.fori_loop(0, KCH, chunk, 0)
        plsc.subcore_barrier()

        for j in range(n_blk):
            r0 = base + j * C
            pltpu.sync_copy(acc_sh.at[pl.ds(r0, C)], rows_v)
            pltpu.sync_copy(rows_v, out_hbm.at[c, pl.ds(r0, C)])

    return seg


def _combine_efeat(pe):
    blocks_per_tile = E_PAD // (L * NW)
    mesh = plsc.VectorSubcoreMesh(core_axis_name="c", subcore_axis_name="s")

    @functools.partial(
        pl.kernel,
        out_type=jax.ShapeDtypeStruct((E_PAD, DA), jnp.float32),
        mesh=mesh,
        scratch_types=[
            pltpu.VMEM((L, DA), jnp.float32),
            pltpu.VMEM((L, DA), jnp.float32),
            pltpu.VMEM((L, DA), jnp.float32),
        ],
    )
    def comb(pe_hbm, out_hbm, p0_v, p1_v, ob_v):
        c = lax.axis_index("c")
        s = lax.axis_index("s")
        wid = c * NS + s
        rows = lax.iota(jnp.int32, L)

        def body(j, carry):
            r0 = (wid * blocks_per_tile + j) * L
            pltpu.sync_copy(pe_hbm.at[0, pl.ds(r0, L)], p0_v)
            pltpu.sync_copy(pe_hbm.at[1, pl.ds(r0, L)], p1_v)
            ccol = jnp.full((L,), D, jnp.int32)
            cnt = (plsc.load_gather(p0_v, [rows, ccol]) +
                   plsc.load_gather(p1_v, [rows, ccol]))
            rcp = 1.0 / jnp.maximum(cnt, 1.0)
            for col in range(D):
                cols = jnp.full((L,), col, jnp.int32)
                g = (plsc.load_gather(p0_v, [rows, cols]) +
                     plsc.load_gather(p1_v, [rows, cols]))
                plsc.store_scatter(ob_v, [rows, cols], g * rcp)
            plsc.store_scatter(ob_v, [rows, ccol], jnp.ones((L,), jnp.float32))
            for col in range(D + 1, DA):
                cols = jnp.full((L,), col, jnp.int32)
                plsc.store_scatter(ob_v, [rows, cols],
                                   jnp.zeros((L,), jnp.float32))
            pltpu.sync_copy(ob_v, out_hbm.at[pl.ds(r0, L)])
            return carry
        lax.fori_loop(0, blocks_per_tile, body, 0)

    return comb(pe)


def _combine_out(pv):
    n_blocks = N_V // L
    max_per_tile = -(-n_blocks // NW)
    mesh = plsc.VectorSubcoreMesh(core_axis_name="c", subcore_axis_name="s")

    @functools.partial(
        pl.kernel,
        out_type=jax.ShapeDtypeStruct((N_V, D), jnp.float32),
        mesh=mesh,
        scratch_types=[
            pltpu.VMEM((L, DA), jnp.float32),
            pltpu.VMEM((L, DA), jnp.float32),
            pltpu.VMEM((L, D), jnp.float32),
        ],
    )
    def comb(pv_hbm, out_hbm, p0_v, p1_v, ob_v):
        c = lax.axis_index("c")
        s = lax.axis_index("s")
        wid = c * NS + s
        rows = lax.iota(jnp.int32, L)

        def body(j, carry):
            b = wid + NW * j

            @pl.when(b < n_blocks)
            def _():
                r0 = b * L
                pltpu.sync_copy(pv_hbm.at[0, pl.ds(r0, L)], p0_v)
                pltpu.sync_copy(pv_hbm.at[1, pl.ds(r0, L)], p1_v)
                ccol = jnp.full((L,), D, jnp.int32)
                cnt = (plsc.load_gather(p0_v, [rows, ccol]) +
                       plsc.load_gather(p1_v, [rows, ccol]))
                rcp = 1.0 / jnp.maximum(cnt, 1.0)
                for col in range(D):
                    cols = jnp.full((L,), col, jnp.int32)
                    g = (plsc.load_gather(p0_v, [rows, cols]) +
                         plsc.load_gather(p1_v, [rows, cols]))
                    plsc.store_scatter(ob_v, [rows, cols], g * rcp)
                pltpu.sync_copy(ob_v, out_hbm.at[pl.ds(r0, L)])
            return carry
        lax.fori_loop(0, max_per_tile, body, 0)

    return comb(pv)


@jax.jit
def _run(X, v_idx, e_idx, W, b):
    xt = _tc_matmul(X, W, b)
    vr = v_idx.reshape(NNZ // C, C)
    er = e_idx.reshape(NNZ // C, C)
    pe = _make_seg_sum(E_PAD)(xt, vr, er)
    ef = _combine_efeat(pe)
    pv = _make_seg_sum(V_PAD)(ef, er, vr)
    return _combine_out(pv)


def kernel(X, v_idx, e_idx, W, b):
    return _run(X, v_idx, e_idx, W, b)


# speedup vs baseline: 7.1193x; 7.1193x over previous
import functools

import jax
import jax.numpy as jnp
from jax import lax
from jax.experimental import pallas as pl
from jax.experimental.pallas import tpu as pltpu
from jax.experimental.pallas import tpu_sc as plsc

N_V = 10000
N_E = 5000
NNZ = 320000
D = 128
DA = 144
E_PAD = 5120
V_PAD = 10240
NC = 2
NS = 16
NW = NC * NS
C = 80
KCH = NNZ // (NW * C)
L = 16


def _tc_matmul(X, W, b):
    blk = 400

    def body(x_ref, w_ref, b_ref, o_ref):
        y = lax.dot_general(x_ref[...], w_ref[...], (((1,), (1,)), ((), ())),
                            preferred_element_type=jnp.float32)
        y = jnp.maximum(y + b_ref[...], 0.0)
        aug = (lax.broadcasted_iota(jnp.int32, (blk, DA - D), 1) == 0)
        o_ref[...] = jnp.concatenate([y, aug.astype(jnp.float32)], axis=1)

    return pl.pallas_call(
        body,
        grid=(N_V // blk,),
        in_specs=[
            pl.BlockSpec((blk, D), lambda i: (i, 0)),
            pl.BlockSpec((D, D), lambda i: (0, 0)),
            pl.BlockSpec((1, D), lambda i: (0, 0)),
        ],
        out_specs=pl.BlockSpec((blk, DA), lambda i: (i, 0)),
        out_shape=jax.ShapeDtypeStruct((N_V, DA), jnp.float32),
    )(X, W, b.reshape(1, D))


def _make_seg_sum(n_pad):
    rows_per_tile = n_pad // NS
    n_blk = rows_per_tile // C
    mesh = plsc.VectorSubcoreMesh(core_axis_name="c", subcore_axis_name="s")

    @functools.partial(
        pl.kernel,
        out_type=jax.ShapeDtypeStruct((NC, n_pad, DA), jnp.float32),
        mesh=mesh,
        compiler_params=pltpu.CompilerParams(use_tc_tiling_on_sc=False),
        scratch_types=[
            pltpu.VMEM((KCH, C), jnp.int32),
            pltpu.VMEM((KCH, C), jnp.int32),
            pltpu.VMEM((C, DA), jnp.float32),
            pltpu.VMEM_SHARED((n_pad, DA), jnp.float32),
            pltpu.SemaphoreType.DMA,
        ],
    )
    def seg(src_hbm, gidx_hbm, sidx_hbm, out_hbm, gidx_v, sidx_v, rows_v,
            acc_sh, sem):
        c = lax.axis_index("c")
        s = lax.axis_index("s")
        wid = c * NS + s

        def zero_row(r, carry):
            for g in range(DA // L):
                rows_v[r, pl.ds(g * L, L)] = jnp.zeros((L,), jnp.float32)
            return carry
        lax.fori_loop(0, C, zero_row, 0)
        base = s * rows_per_tile
        for j in range(n_blk):
            pltpu.sync_copy(rows_v, acc_sh.at[pl.ds(base + j * C, C)])
        plsc.subcore_barrier()

        pltpu.sync_copy(gidx_hbm.at[wid], gidx_v)
        pltpu.sync_copy(sidx_hbm.at[wid], sidx_v)

        def chunk(k, carry):
            pltpu.async_copy(src_hbm.at[gidx_v.at[k]], rows_v, sem).wait()
            pltpu.sync_copy(rows_v, acc_sh.at[sidx_v.at[k]], add=True)
            return carry
        lax.fori_loop(0, KCH, chunk, 0)
        plsc.subcore_barrier()

        for j in range(n_blk):
            r0 = base + j * C
            pltpu.sync_copy(acc_sh.at[pl.ds(r0, C)], rows_v)
            pltpu.sync_copy(rows_v, out_hbm.at[c, pl.ds(r0, C)])

    return seg


def _combine_efeat(pe):
    blk = 512

    def body(pe_ref, o_ref):
        y = pe_ref[0] + pe_ref[1]
        scale = 1.0 / jnp.maximum(y[:, D:D + 1], 1.0)
        aug = (lax.broadcasted_iota(jnp.int32, (blk, DA - D), 1) == 0)
        o_ref[...] = jnp.concatenate(
            [y[:, :D] * scale, aug.astype(jnp.float32)], axis=1)

    return pl.pallas_call(
        body,
        grid=(E_PAD // blk,),
        in_specs=[pl.BlockSpec((NC, blk, DA), lambda i: (0, i, 0))],
        out_specs=pl.BlockSpec((blk, DA), lambda i: (i, 0)),
        out_shape=jax.ShapeDtypeStruct((E_PAD, DA), jnp.float32),
    )(pe)


def _combine_out(pv):
    blk = 400

    def body(pv_ref, o_ref):
        y = pv_ref[0] + pv_ref[1]
        scale = 1.0 / jnp.maximum(y[:, D:D + 1], 1.0)
        o_ref[...] = y[:, :D] * scale

    return pl.pallas_call(
        body,
        grid=(N_V // blk,),
        in_specs=[pl.BlockSpec((NC, blk, DA), lambda i: (0, i, 0))],
        out_specs=pl.BlockSpec((blk, D), lambda i: (i, 0)),
        out_shape=jax.ShapeDtypeStruct((N_V, D), jnp.float32),
    )(pv)


@jax.jit
def _run(X, v_idx, e_idx, W, b):
    xt = _tc_matmul(X, W, b)
    vr = v_idx.reshape(NW, KCH, C)
    er = e_idx.reshape(NW, KCH, C)
    pe = _make_seg_sum(E_PAD)(xt, vr, er)
    ef = _combine_efeat(pe)
    pv = _make_seg_sum(V_PAD)(ef, er, vr)
    return _combine_out(pv)


def kernel(X, v_idx, e_idx, W, b):
    return _run(X, v_idx, e_idx, W, b)


# speedup vs baseline: 9.7056x; 1.3633x over previous
import functools

import jax
import jax.numpy as jnp
from jax import lax
from jax.experimental import pallas as pl
from jax.experimental.pallas import tpu as pltpu
from jax.experimental.pallas import tpu_sc as plsc

N_V = 10000
N_E = 5000
NNZ = 320000
D = 128
DA = 144
E_PAD = 5120
V_PAD = 10240
NC = 2
NS = 16
NW = NC * NS
PPT = NNZ // NW
L = 16


def _tc_matmul(X, W, b):
    blk = 400

    def body(x_ref, w_ref, b_ref, o_ref):
        y = lax.dot_general(x_ref[...], w_ref[...], (((1,), (1,)), ((), ())),
                            preferred_element_type=jnp.float32)
        y = jnp.maximum(y + b_ref[...], 0.0)
        aug = (lax.broadcasted_iota(jnp.int32, (blk, DA - D), 1) == 0)
        o_ref[...] = jnp.concatenate([y, aug.astype(jnp.float32)], axis=1)

    return pl.pallas_call(
        body,
        grid=(N_V // blk,),
        in_specs=[
            pl.BlockSpec((blk, D), lambda i: (i, 0)),
            pl.BlockSpec((D, D), lambda i: (0, 0)),
            pl.BlockSpec((1, D), lambda i: (0, 0)),
        ],
        out_specs=pl.BlockSpec((blk, DA), lambda i: (i, 0)),
        out_shape=jax.ShapeDtypeStruct((N_V, DA), jnp.float32),
    )(X, W, b.reshape(1, D))


def _make_seg_sum(n_pad, C):
    KCH = PPT // C
    rows_per_tile = n_pad // NS
    n_blk = rows_per_tile // C
    mesh = plsc.VectorSubcoreMesh(core_axis_name="c", subcore_axis_name="s")

    @functools.partial(
        pl.kernel,
        out_type=jax.ShapeDtypeStruct((NC, n_pad, DA), jnp.float32),
        mesh=mesh,
        compiler_params=pltpu.CompilerParams(use_tc_tiling_on_sc=False),
        scratch_types=[
            pltpu.VMEM((KCH, C), jnp.int32),
            pltpu.VMEM((KCH, C), jnp.int32),
            pltpu.VMEM((C, DA), jnp.float32),
            pltpu.VMEM((C, DA), jnp.float32),
            pltpu.VMEM_SHARED((n_pad, DA), jnp.float32),
            pltpu.SemaphoreType.DMA,
            pltpu.SemaphoreType.DMA,
        ],
    )
    def seg(src_hbm, gidx_hbm, sidx_hbm, out_hbm, gidx_v, sidx_v, rows0_v,
            rows1_v, acc_sh, sem0, sem1):
        c = lax.axis_index("c")
        s = lax.axis_index("s")
        wid = c * NS + s
        rows_b = (rows0_v, rows1_v)
        sem_b = (sem0, sem1)

        def zero_row(r, carry):
            for g in range(DA // L):
                rows0_v[r, pl.ds(g * L, L)] = jnp.zeros((L,), jnp.float32)
            return carry
        lax.fori_loop(0, C, zero_row, 0)
        base = s * rows_per_tile
        for j in range(n_blk):
            pltpu.sync_copy(rows0_v, acc_sh.at[pl.ds(base + j * C, C)])
        plsc.subcore_barrier()

        pltpu.sync_copy(gidx_hbm.at[wid], gidx_v)
        pltpu.sync_copy(sidx_hbm.at[wid], sidx_v)

        for b in range(2):
            pltpu.async_copy(src_hbm.at[gidx_v.at[b]], rows_b[b], sem_b[b])

        def chunk(i, carry):
            for b in range(2):
                k = 2 * i + b
                pltpu.make_async_copy(
                    src_hbm.at[gidx_v.at[k]], rows_b[b], sem_b[b]).wait()
                pltpu.sync_copy(rows_b[b], acc_sh.at[sidx_v.at[k]], add=True)

                @pl.when(k + 2 < KCH)
                def _():
                    pltpu.async_copy(
                        src_hbm.at[gidx_v.at[k + 2]], rows_b[b], sem_b[b])
            return carry
        lax.fori_loop(0, KCH // 2, chunk, 0)
        if KCH % 2:
            k = KCH - 1
            pltpu.make_async_copy(
                src_hbm.at[gidx_v.at[k]], rows0_v, sem0).wait()
            pltpu.sync_copy(rows0_v, acc_sh.at[sidx_v.at[k]], add=True)
        plsc.subcore_barrier()

        for j in range(n_blk):
            r0 = base + j * C
            pltpu.sync_copy(acc_sh.at[pl.ds(r0, C)], out_hbm.at[c, pl.ds(r0, C)])

    return seg


def _combine_efeat(pe):
    blk = 512

    def body(pe_ref, o_ref):
        y = pe_ref[0] + pe_ref[1]
        scale = 1.0 / jnp.maximum(y[:, D:D + 1], 1.0)
        aug = (lax.broadcasted_iota(jnp.int32, (blk, DA - D), 1) == 0)
        o_ref[...] = jnp.concatenate(
            [y[:, :D] * scale, aug.astype(jnp.float32)], axis=1)

    return pl.pallas_call(
        body,
        grid=(E_PAD // blk,),
        in_specs=[pl.BlockSpec((NC, blk, DA), lambda i: (0, i, 0))],
        out_specs=pl.BlockSpec((blk, DA), lambda i: (i, 0)),
        out_shape=jax.ShapeDtypeStruct((E_PAD, DA), jnp.float32),
    )(pe)


def _combine_out(pv):
    blk = 400

    def body(pv_ref, o_ref):
        y = pv_ref[0] + pv_ref[1]
        scale = 1.0 / jnp.maximum(y[:, D:D + 1], 1.0)
        o_ref[...] = y[:, :D] * scale

    return pl.pallas_call(
        body,
        grid=(N_V // blk,),
        in_specs=[pl.BlockSpec((NC, blk, DA), lambda i: (0, i, 0))],
        out_specs=pl.BlockSpec((blk, D), lambda i: (i, 0)),
        out_shape=jax.ShapeDtypeStruct((N_V, D), jnp.float32),
    )(pv)


@jax.jit
def _run(X, v_idx, e_idx, W, b):
    xt = _tc_matmul(X, W, b)
    pe = _make_seg_sum(E_PAD, 80)(
        xt, v_idx.reshape(NW, PPT // 80, 80), e_idx.reshape(NW, PPT // 80, 80))
    ef = _combine_efeat(pe)
    pv = _make_seg_sum(V_PAD, 40)(
        ef, e_idx.reshape(NW, PPT // 40, 40), v_idx.reshape(NW, PPT // 40, 40))
    return _combine_out(pv)


def kernel(X, v_idx, e_idx, W, b):
    return _run(X, v_idx, e_idx, W, b)


# speedup vs baseline: 13.5414x; 1.3952x over previous
import functools

import jax
import jax.numpy as jnp
from jax import lax
from jax.experimental import pallas as pl
from jax.experimental.pallas import tpu as pltpu
from jax.experimental.pallas import tpu_sc as plsc

N_V = 10000
N_E = 5000
NNZ = 320000
D = 128
E_PAD = 5120
V_PAD = 10240
NC = 2
NS = 16
NW = NC * NS
PPT = NNZ // NW
C = 80
KCH = PPT // C
L = 16


def _sc_counts(v_idx, e_idx):
    mesh = plsc.VectorSubcoreMesh(core_axis_name="c", subcore_axis_name="s")

    @functools.partial(
        pl.kernel,
        out_type=(jax.ShapeDtypeStruct((NW, E_PAD), jnp.float32),
                  jax.ShapeDtypeStruct((NW, V_PAD), jnp.float32)),
        mesh=mesh,
        compiler_params=pltpu.CompilerParams(
            use_tc_tiling_on_sc=False, needs_layout_passes=False),
        scratch_types=[
            pltpu.VMEM((PPT,), jnp.int32),
            pltpu.VMEM((PPT,), jnp.int32),
            pltpu.VMEM((E_PAD,), jnp.float32),
            pltpu.VMEM((V_PAD,), jnp.float32),
        ],
    )
    def hist(vidx_hbm, eidx_hbm, oute_hbm, outv_hbm, vidx_v, eidx_v,
             ehist_v, vhist_v):
        c = lax.axis_index("c")
        s = lax.axis_index("s")
        wid = c * NS + s
        zeros = jnp.zeros((L,), jnp.float32)
        ones = jnp.ones((L,), jnp.float32)

        def ze(r, carry):
            ehist_v[pl.ds(r * L, L)] = zeros
            return carry
        lax.fori_loop(0, E_PAD // L, ze, 0)

        def zv(r, carry):
            vhist_v[pl.ds(r * L, L)] = zeros
            return carry
        lax.fori_loop(0, V_PAD // L, zv, 0)

        pltpu.sync_copy(vidx_hbm.at[pl.ds(wid * PPT, PPT)], vidx_v)
        pltpu.sync_copy(eidx_hbm.at[pl.ds(wid * PPT, PPT)], eidx_v)

        def acc(t, carry):
            v16 = vidx_v[pl.ds(t * L, L)]
            e16 = eidx_v[pl.ds(t * L, L)]
            plsc.addupdate_scatter(vhist_v, [v16], ones)
            plsc.addupdate_scatter(ehist_v, [e16], ones)
            return carry
        lax.fori_loop(0, PPT // L, acc, 0)

        pltpu.sync_copy(ehist_v, oute_hbm.at[wid])
        pltpu.sync_copy(vhist_v, outv_hbm.at[wid])

    return hist(v_idx, e_idx)


def _tc_matmul(X, W, b):
    blk = 400

    def body(x_ref, w_ref, b_ref, o_ref):
        y = lax.dot_general(x_ref[...], w_ref[...], (((1,), (1,)), ((), ())),
                            preferred_element_type=jnp.float32)
        o_ref[...] = jnp.maximum(y + b_ref[...], 0.0)

    return pl.pallas_call(
        body,
        grid=(N_V // blk,),
        in_specs=[
            pl.BlockSpec((blk, D), lambda i: (i, 0)),
            pl.BlockSpec((D, D), lambda i: (0, 0)),
            pl.BlockSpec((1, D), lambda i: (0, 0)),
        ],
        out_specs=pl.BlockSpec((blk, D), lambda i: (i, 0)),
        out_shape=jax.ShapeDtypeStruct((N_V, D), jnp.float32),
    )(X, W, b.reshape(1, D))


def _make_seg_sum(n_pad, nbuf=2):
    rows_per_tile = n_pad // NS
    n_blk = rows_per_tile // C
    ni = 2 * nbuf
    mesh = plsc.VectorSubcoreMesh(core_axis_name="c", subcore_axis_name="s")

    @functools.partial(
        pl.kernel,
        out_type=jax.ShapeDtypeStruct((NC, n_pad, D), jnp.float32),
        mesh=mesh,
        scratch_types=(
            [pltpu.VMEM((C,), jnp.int32) for _ in range(ni)] +
            [pltpu.VMEM((C,), jnp.int32) for _ in range(ni)] +
            [pltpu.VMEM((C, D), jnp.float32) for _ in range(nbuf)] +
            [pltpu.VMEM_SHARED((n_pad, D), jnp.float32)] +
            [pltpu.SemaphoreType.DMA for _ in range(ni)] +
            [pltpu.SemaphoreType.DMA for _ in range(nbuf)]
        ),
    )
    def seg(src_hbm, gidx_hbm, sidx_hbm, out_hbm, *bufs):
        gi_b = bufs[:ni]
        si_b = bufs[ni:2 * ni]
        rows_b = bufs[2 * ni:2 * ni + nbuf]
        acc_sh = bufs[2 * ni + nbuf]
        isem = bufs[2 * ni + nbuf + 1:2 * ni + nbuf + 1 + ni]
        gsem = bufs[2 * ni + nbuf + 1 + ni:]
        c = lax.axis_index("c")
        s = lax.axis_index("s")
        wid = c * NS + s
        p0 = wid * PPT

        def idx_start(k, slot):
            pltpu.async_copy(
                gidx_hbm.at[pl.ds(p0 + k * C, C)], gi_b[slot], isem[slot])
            pltpu.async_copy(
                sidx_hbm.at[pl.ds(p0 + k * C, C)], si_b[slot], isem[slot])

        def idx_wait(slot):
            pltpu.make_async_copy(
                gidx_hbm.at[pl.ds(0, C)], gi_b[slot], isem[slot]).wait()
            pltpu.make_async_copy(
                sidx_hbm.at[pl.ds(0, C)], si_b[slot], isem[slot]).wait()

        def gather_start(slot, b):
            pltpu.async_copy(src_hbm.at[gi_b[slot]], rows_b[b], gsem[b])

        def gather_wait(b):
            pltpu.make_async_copy(
                src_hbm.at[gi_b[0]], rows_b[b], gsem[b]).wait()

        for j in range(min(ni, KCH)):
            idx_start(j, j)

        def zero_row(r, carry):
            for g in range(D // L):
                rows_b[0][r, pl.ds(g * L, L)] = jnp.zeros((L,), jnp.float32)
            return carry
        lax.fori_loop(0, C, zero_row, 0)
        base = s * rows_per_tile
        for j in range(n_blk):
            pltpu.sync_copy(rows_b[0], acc_sh.at[pl.ds(base + j * C, C)])
        plsc.subcore_barrier()

        for b in range(min(nbuf, KCH)):
            idx_wait(b)
            gather_start(b, b)

        def group(jg, carry):
            k0 = ni * jg
            for t in range(ni):
                k = k0 + t
                b = t % nbuf
                gather_wait(b)
                pltpu.sync_copy(rows_b[b], acc_sh.at[si_b[t]], add=True)
                nt = (t + nbuf) % ni

                @pl.when(k + nbuf < KCH)
                def _():
                    idx_wait(nt)
                    gather_start(nt, b)

                @pl.when(k + ni < KCH)
                def _():
                    idx_start(k + ni, t)
            return carry
        lax.fori_loop(0, KCH // ni, group, 0)
        for t in range(KCH % ni):
            k = (KCH // ni) * ni + t
            b = t % nbuf
            gather_wait(b)
            pltpu.sync_copy(rows_b[b], acc_sh.at[si_b[t]], add=True)
            if k + nbuf < KCH:
                nt = (t + nbuf) % ni
                idx_wait(nt)
                gather_start(nt, b)
        plsc.subcore_barrier()

        for j in range(n_blk):
            r0 = base + j * C
            pltpu.sync_copy(acc_sh.at[pl.ds(r0, C)], out_hbm.at[c, pl.ds(r0, C)])

    return seg


def _combine_efeat(pe, he):
    blk = 512

    def body(pe_ref, he_ref, o_ref):
        y = pe_ref[0] + pe_ref[1]
        cnt = jnp.sum(he_ref[...], axis=0)
        o_ref[...] = y * (1.0 / jnp.maximum(cnt, 1.0))[:, None]

    return pl.pallas_call(
        body,
        grid=(E_PAD // blk,),
        in_specs=[pl.BlockSpec((NC, blk, D), lambda i: (0, i, 0)),
                  pl.BlockSpec((NW, blk), lambda i: (0, i))],
        out_specs=pl.BlockSpec((blk, D), lambda i: (i, 0)),
        out_shape=jax.ShapeDtypeStruct((E_PAD, D), jnp.float32),
    )(pe, he)


def _combine_out(pv, hv):
    blk = 512

    def body(pv_ref, hv_ref, o_ref):
        y = pv_ref[0] + pv_ref[1]
        cnt = jnp.sum(hv_ref[...], axis=0)
        o_ref[...] = y * (1.0 / jnp.maximum(cnt, 1.0))[:, None]

    return pl.pallas_call(
        body,
        grid=(V_PAD // blk,),
        in_specs=[pl.BlockSpec((NC, blk, D), lambda i: (0, i, 0)),
                  pl.BlockSpec((NW, blk), lambda i: (0, i))],
        out_specs=pl.BlockSpec((blk, D), lambda i: (i, 0)),
        out_shape=jax.ShapeDtypeStruct((V_PAD, D), jnp.float32),
    )(pv, hv)


@jax.jit
def _run(X, v_idx, e_idx, W, b):
    he, hv = _sc_counts(v_idx, e_idx)
    xt = _tc_matmul(X, W, b)
    pe = _make_seg_sum(E_PAD)(xt, v_idx, e_idx)
    ef = _combine_efeat(pe, he)
    pv = _make_seg_sum(V_PAD)(ef, e_idx, v_idx)
    return _combine_out(pv, hv)[:N_V]


def kernel(X, v_idx, e_idx, W, b):
    return _run(X, v_idx, e_idx, W, b)


# speedup vs baseline: 16.4765x; 1.2168x over previous
import functools

import jax
import jax.numpy as jnp
from jax import lax
from jax.experimental import pallas as pl
from jax.experimental.pallas import tpu as pltpu
from jax.experimental.pallas import tpu_sc as plsc

N_V = 10000
N_E = 5000
NNZ = 320000
D = 128
E_PAD = 5120
V_PAD = 10240
NC = 2
NS = 16
NW = NC * NS
PPT = NNZ // NW
C = 80
KCH = PPT // C
L = 16


def _sc_counts(v_idx, e_idx):
    mesh = plsc.VectorSubcoreMesh(core_axis_name="c", subcore_axis_name="s")

    @functools.partial(
        pl.kernel,
        out_type=(jax.ShapeDtypeStruct((NW, E_PAD), jnp.float32),
                  jax.ShapeDtypeStruct((NW, V_PAD), jnp.float32)),
        mesh=mesh,
        compiler_params=pltpu.CompilerParams(
            use_tc_tiling_on_sc=False, needs_layout_passes=False),
        scratch_types=[
            pltpu.VMEM((PPT,), jnp.int32),
            pltpu.VMEM((PPT,), jnp.int32),
            pltpu.VMEM((E_PAD,), jnp.float32),
            pltpu.VMEM((V_PAD,), jnp.float32),
        ],
    )
    def hist(vidx_hbm, eidx_hbm, oute_hbm, outv_hbm, vidx_v, eidx_v,
             ehist_v, vhist_v):
        c = lax.axis_index("c")
        s = lax.axis_index("s")
        wid = c * NS + s
        zeros = jnp.zeros((L,), jnp.float32)
        ones = jnp.ones((L,), jnp.float32)

        def ze(r, carry):
            ehist_v[pl.ds(r * L, L)] = zeros
            return carry
        lax.fori_loop(0, E_PAD // L, ze, 0)

        def zv(r, carry):
            vhist_v[pl.ds(r * L, L)] = zeros
            return carry
        lax.fori_loop(0, V_PAD // L, zv, 0)

        pltpu.sync_copy(vidx_hbm.at[pl.ds(wid * PPT, PPT)], vidx_v)
        pltpu.sync_copy(eidx_hbm.at[pl.ds(wid * PPT, PPT)], eidx_v)

        def acc(t, carry):
            v16 = vidx_v[pl.ds(t * L, L)]
            e16 = eidx_v[pl.ds(t * L, L)]
            plsc.addupdate_scatter(vhist_v, [v16], ones)
            plsc.addupdate_scatter(ehist_v, [e16], ones)
            return carry
        lax.fori_loop(0, PPT // L, acc, 0)

        pltpu.sync_copy(ehist_v, oute_hbm.at[wid])
        pltpu.sync_copy(vhist_v, outv_hbm.at[wid])

    return hist(v_idx, e_idx)


def _tc_matmul(X, W, b):
    blk = 400

    def body(x_ref, w_ref, b_ref, o_ref):
        y = lax.dot_general(x_ref[...], w_ref[...], (((1,), (1,)), ((), ())),
                            preferred_element_type=jnp.float32)
        o_ref[...] = jnp.maximum(y + b_ref[...], 0.0)

    return pl.pallas_call(
        body,
        grid=(N_V // blk,),
        in_specs=[
            pl.BlockSpec((blk, D), lambda i: (i, 0)),
            pl.BlockSpec((D, D), lambda i: (0, 0)),
            pl.BlockSpec((1, D), lambda i: (0, 0)),
        ],
        out_specs=pl.BlockSpec((blk, D), lambda i: (i, 0)),
        out_shape=jax.ShapeDtypeStruct((N_V, D), jnp.float32),
    )(X, W, b.reshape(1, D))


def _make_seg_sum(n_pad, nbuf=4):
    rows_per_tile = n_pad // NS
    n_blk = rows_per_tile // C
    ni = 2 * nbuf
    mesh = plsc.VectorSubcoreMesh(core_axis_name="c", subcore_axis_name="s")

    @functools.partial(
        pl.kernel,
        out_type=jax.ShapeDtypeStruct((NC, n_pad, D), jnp.float32),
        mesh=mesh,
        scratch_types=(
            [pltpu.VMEM((C,), jnp.int32) for _ in range(ni)] +
            [pltpu.VMEM((C,), jnp.int32) for _ in range(ni)] +
            [pltpu.VMEM((C, D), jnp.float32) for _ in range(nbuf)] +
            [pltpu.VMEM_SHARED((n_pad, D), jnp.float32)] +
            [pltpu.SemaphoreType.DMA for _ in range(ni)] +
            [pltpu.SemaphoreType.DMA for _ in range(nbuf)]
        ),
    )
    def seg(src_hbm, gidx_hbm, sidx_hbm, out_hbm, *bufs):
        gi_b = bufs[:ni]
        si_b = bufs[ni:2 * ni]
        rows_b = bufs[2 * ni:2 * ni + nbuf]
        acc_sh = bufs[2 * ni + nbuf]
        isem = bufs[2 * ni + nbuf + 1:2 * ni + nbuf + 1 + ni]
        gsem = bufs[2 * ni + nbuf + 1 + ni:]
        c = lax.axis_index("c")
        s = lax.axis_index("s")
        wid = c * NS + s
        p0 = wid * PPT

        def idx_start(k, slot):
            pltpu.async_copy(
                gidx_hbm.at[pl.ds(p0 + k * C, C)], gi_b[slot], isem[slot])
            pltpu.async_copy(
                sidx_hbm.at[pl.ds(p0 + k * C, C)], si_b[slot], isem[slot])

        def idx_wait(slot):
            pltpu.make_async_copy(
                gidx_hbm.at[pl.ds(0, C)], gi_b[slot], isem[slot]).wait()
            pltpu.make_async_copy(
                sidx_hbm.at[pl.ds(0, C)], si_b[slot], isem[slot]).wait()

        def gather_start(slot, b):
            pltpu.async_copy(src_hbm.at[gi_b[slot]], rows_b[b], gsem[b])

        def gather_wait(b):
            pltpu.make_async_copy(
                src_hbm.at[gi_b[0]], rows_b[b], gsem[b]).wait()

        for j in range(min(ni, KCH)):
            idx_start(j, j)

        def zero_row(r, carry):
            for g in range(D // L):
                rows_b[0][r, pl.ds(g * L, L)] = jnp.zeros((L,), jnp.float32)
            return carry
        lax.fori_loop(0, C, zero_row, 0)
        base = s * rows_per_tile
        for j in range(n_blk):
            pltpu.sync_copy(rows_b[0], acc_sh.at[pl.ds(base + j * C, C)])
        plsc.subcore_barrier()

        for b in range(min(nbuf, KCH)):
            idx_wait(b)
            gather_start(b, b)

        def group(jg, carry):
            k0 = ni * jg
            for t in range(ni):
                k = k0 + t
                b = t % nbuf
                gather_wait(b)
                pltpu.sync_copy(rows_b[b], acc_sh.at[si_b[t]], add=True)
                nt = (t + nbuf) % ni

                @pl.when(k + nbuf < KCH)
                def _():
                    idx_wait(nt)
                    gather_start(nt, b)

                @pl.when(k + ni < KCH)
                def _():
                    idx_start(k + ni, t)
            return carry
        lax.fori_loop(0, KCH // ni, group, 0)
        for t in range(KCH % ni):
            k = (KCH // ni) * ni + t
            b = t % nbuf
            gather_wait(b)
            pltpu.sync_copy(rows_b[b], acc_sh.at[si_b[t]], add=True)
            if k + nbuf < KCH:
                nt = (t + nbuf) % ni
                idx_wait(nt)
                gather_start(nt, b)
        plsc.subcore_barrier()

        for j in range(n_blk):
            r0 = base + j * C
            pltpu.sync_copy(acc_sh.at[pl.ds(r0, C)], out_hbm.at[c, pl.ds(r0, C)])

    return seg


def _combine_efeat(pe, he):
    blk = 512

    def body(pe_ref, he_ref, o_ref):
        y = pe_ref[0] + pe_ref[1]
        cnt = jnp.sum(he_ref[...], axis=0)
        o_ref[...] = y * (1.0 / jnp.maximum(cnt, 1.0))[:, None]

    return pl.pallas_call(
        body,
        grid=(E_PAD // blk,),
        in_specs=[pl.BlockSpec((NC, blk, D), lambda i: (0, i, 0)),
                  pl.BlockSpec((NW, blk), lambda i: (0, i))],
        out_specs=pl.BlockSpec((blk, D), lambda i: (i, 0)),
        out_shape=jax.ShapeDtypeStruct((E_PAD, D), jnp.float32),
    )(pe, he)


def _combine_out(pv, hv):
    blk = 512

    def body(pv_ref, hv_ref, o_ref):
        y = pv_ref[0] + pv_ref[1]
        cnt = jnp.sum(hv_ref[...], axis=0)
        o_ref[...] = y * (1.0 / jnp.maximum(cnt, 1.0))[:, None]

    return pl.pallas_call(
        body,
        grid=(V_PAD // blk,),
        in_specs=[pl.BlockSpec((NC, blk, D), lambda i: (0, i, 0)),
                  pl.BlockSpec((NW, blk), lambda i: (0, i))],
        out_specs=pl.BlockSpec((blk, D), lambda i: (i, 0)),
        out_shape=jax.ShapeDtypeStruct((V_PAD, D), jnp.float32),
    )(pv, hv)


@jax.jit
def _run(X, v_idx, e_idx, W, b):
    he, hv = _sc_counts(v_idx, e_idx)
    xt = _tc_matmul(X, W, b)
    pe = _make_seg_sum(E_PAD)(xt, v_idx, e_idx)
    ef = _combine_efeat(pe, he)
    pv = _make_seg_sum(V_PAD)(ef, e_idx, v_idx)
    return _combine_out(pv, hv)[:N_V]


def kernel(X, v_idx, e_idx, W, b):
    return _run(X, v_idx, e_idx, W, b)
